# baseline (speedup 1.0000x reference)
import functools

import jax
import jax.numpy as jnp
from jax import lax
from jax.experimental import pallas as pl
from jax.experimental.pallas import tpu as pltpu
from jax.experimental.pallas import tpu_sc as plsc

BATCH = 16384
EMBED_DIM = 64
NUM_CORES = 2
NUM_SUBCORES = 16
NUM_WORKERS = NUM_CORES * NUM_SUBCORES
B_PER_W = BATCH // NUM_WORKERS


@functools.cache
def _build():
    mesh = plsc.VectorSubcoreMesh(
        core_axis_name="c", subcore_axis_name="s",
        num_cores=NUM_CORES, num_subcores=NUM_SUBCORES)

    @functools.partial(
        pl.kernel,
        mesh=mesh,
        compiler_params=pltpu.CompilerParams(use_tc_tiling_on_sc=False),
        out_type=(
            jax.ShapeDtypeStruct((BATCH, EMBED_DIM), jnp.float32),
            jax.ShapeDtypeStruct((BATCH, EMBED_DIM), jnp.float32),
        ),
        scratch_types=[
            pltpu.VMEM((B_PER_W,), jnp.int32),
            pltpu.VMEM((B_PER_W,), jnp.int32),
            pltpu.VMEM((B_PER_W, EMBED_DIM), jnp.float32),
            pltpu.VMEM((B_PER_W, EMBED_DIM), jnp.float32),
            pltpu.SemaphoreType.DMA,
            pltpu.SemaphoreType.DMA,
        ],
    )
    def gather2(qu_hbm, qi_hbm, ut_hbm, it_hbm, out_u, out_i,
                uidx_v, iidx_v, urows_v, irows_v, usem, isem):
        wid = lax.axis_index("s") * NUM_CORES + lax.axis_index("c")
        base = wid * B_PER_W
        pltpu.sync_copy(qu_hbm.at[pl.ds(base, B_PER_W)], uidx_v)
        pltpu.sync_copy(qi_hbm.at[pl.ds(base, B_PER_W)], iidx_v)
        cu = pltpu.async_copy(ut_hbm.at[uidx_v], urows_v, usem)
        ci = pltpu.async_copy(it_hbm.at[iidx_v], irows_v, isem)
        cu.wait()
        pltpu.sync_copy(urows_v, out_u.at[pl.ds(base, B_PER_W)])
        ci.wait()
        pltpu.sync_copy(irows_v, out_i.at[pl.ds(base, B_PER_W)])

    return gather2


def kernel(query_users, query_items, user_table, item_table):
    if query_users.ndim > 1:
        query_users = jnp.squeeze(query_users, axis=0)
    if query_items.ndim > 1:
        query_items = jnp.squeeze(query_items, axis=0)
    return _build()(query_users.astype(jnp.int32),
                    query_items.astype(jnp.int32),
                    user_table, item_table)


# speedup vs baseline: 1.5445x; 1.5445x over previous
import functools

import jax
import jax.numpy as jnp
from jax import lax
from jax.experimental import pallas as pl
from jax.experimental.pallas import tpu as pltpu
from jax.experimental.pallas import tpu_sc as plsc

BATCH = 16384
EMBED_DIM = 64
NUM_CORES = 2
NUM_SUBCORES = 16
NUM_WORKERS = NUM_CORES * NUM_SUBCORES
B_PER_W = BATCH // NUM_WORKERS
CHUNK = 64
N_CHUNKS = B_PER_W // CHUNK


@functools.cache
def _build():
    mesh = plsc.VectorSubcoreMesh(
        core_axis_name="c", subcore_axis_name="s",
        num_cores=NUM_CORES, num_subcores=NUM_SUBCORES)

    @functools.partial(
        pl.kernel,
        mesh=mesh,
        out_type=(
            jax.ShapeDtypeStruct((BATCH, EMBED_DIM), jnp.float32),
            jax.ShapeDtypeStruct((BATCH, EMBED_DIM), jnp.float32),
        ),
        scratch_types=[
            pltpu.VMEM((2 * B_PER_W + 16,), jnp.int32),
            pltpu.VMEM((2, CHUNK, EMBED_DIM), jnp.float32),
            pltpu.SemaphoreType.DMA,
            pltpu.SemaphoreType.DMA,
        ],
    )
    def gather2(qu_hbm, qi_hbm, ut_hbm, it_hbm, out_u, out_i,
                idx_v, rows_v, gsem, wsem):
        wid = lax.axis_index("s") * NUM_CORES + lax.axis_index("c")
        base = wid * B_PER_W
        pltpu.sync_copy(qu_hbm.at[pl.ds(base, B_PER_W)],
                        idx_v.at[pl.ds(0, B_PER_W)])
        pltpu.sync_copy(qi_hbm.at[pl.ds(base, B_PER_W)],
                        idx_v.at[pl.ds(B_PER_W, B_PER_W)])

        def fire_gathers(tbl, ioff, buf):
            def body(j, _):
                q = idx_v[pl.ds(ioff + j, 16)][0]
                pltpu.async_copy(tbl.at[pl.ds(q, 1)],
                                 rows_v.at[buf, pl.ds(j, 1)], gsem)
                return ()
            lax.fori_loop(0, CHUNK, body, ())

        def fire_writes(out, ooff, buf):
            def body(j, _):
                pltpu.async_copy(rows_v.at[buf, pl.ds(j, 1)],
                                 out.at[pl.ds(ooff + j, 1)], wsem)
                return ()
            lax.fori_loop(0, CHUNK, body, ())

        def drain(sem):
            pltpu.make_async_copy(
                ut_hbm.at[pl.ds(0, CHUNK)], rows_v.at[0], sem).wait()

        plan = [(ut_hbm, out_u, 0)] * N_CHUNKS + [(it_hbm, out_i, 1)] * N_CHUNKS
        fire_gathers(plan[0][0], 0, 0)
        for c in range(2 * N_CHUNKS):
            tbl, out, t = plan[c]
            coff = (c - t * N_CHUNKS) * CHUNK
            drain(gsem)
            if c >= 1:
                drain(wsem)
            if c + 1 < 2 * N_CHUNKS:
                tbl2, _, t2 = plan[c + 1]
                coff2 = (c + 1 - t2 * N_CHUNKS) * CHUNK
                fire_gathers(tbl2, t2 * B_PER_W + coff2, (c + 1) % 2)
            fire_writes(out, base + coff, c % 2)
        drain(wsem)

    return gather2


def kernel(query_users, query_items, user_table, item_table):
    if query_users.ndim > 1:
        query_users = jnp.squeeze(query_users, axis=0)
    if query_items.ndim > 1:
        query_items = jnp.squeeze(query_items, axis=0)
    return _build()(query_users.astype(jnp.int32),
                    query_items.astype(jnp.int32),
                    user_table, item_table)


# speedup vs baseline: 1.5720x; 1.0179x over previous
import functools

import jax
import jax.numpy as jnp
from jax import lax
from jax.experimental import pallas as pl
from jax.experimental.pallas import tpu as pltpu
from jax.experimental.pallas import tpu_sc as plsc

BATCH = 16384
EMBED_DIM = 64
OUT_LANES = 128
NUM_CORES = 2
NUM_SUBCORES = 16
NUM_WORKERS = NUM_CORES * NUM_SUBCORES
B_PER_W = BATCH // NUM_WORKERS
WIN = 128
N_WIN = B_PER_W // WIN
LANES = 16


@functools.cache
def _build():
    mesh = plsc.VectorSubcoreMesh(
        core_axis_name="c", subcore_axis_name="s",
        num_cores=NUM_CORES, num_subcores=NUM_SUBCORES)

    @functools.partial(
        pl.kernel,
        mesh=mesh,
        out_type=(
            jax.ShapeDtypeStruct((BATCH, EMBED_DIM), jnp.float32),
            jax.ShapeDtypeStruct((BATCH, EMBED_DIM), jnp.float32),
        ),
        scratch_types=[
            pltpu.VMEM((2 * B_PER_W,), jnp.int32),
            pltpu.VMEM((B_PER_W, EMBED_DIM), jnp.float32),
            pltpu.SemaphoreType.DMA,
            pltpu.SemaphoreType.DMA,
            pltpu.SemaphoreType.DMA,
        ],
    )
    def gather2(qu_hbm, qi_hbm, ut_hbm, it_hbm, out_u, out_i,
                idx_v, rows_v, gsem_a, gsem_b, wsem):
        wid = lax.axis_index("s") * NUM_CORES + lax.axis_index("c")
        base = wid * B_PER_W
        pltpu.sync_copy(qu_hbm.at[pl.ds(base, B_PER_W)],
                        idx_v.at[pl.ds(0, B_PER_W)])
        pltpu.sync_copy(qi_hbm.at[pl.ds(base, B_PER_W)],
                        idx_v.at[pl.ds(B_PER_W, B_PER_W)])

        def fire_gathers(tbl, ioff, w, sem):
            def group(g, _):
                off = w * WIN + g * LANES
                v = idx_v[pl.ds(ioff + off, LANES)]
                for lane in range(LANES):
                    pltpu.async_copy(
                        tbl.at[pl.ds(v[lane], 1)],
                        rows_v.at[pl.ds(off + lane, 1)],
                        sem)
                return ()
            lax.fori_loop(0, WIN // LANES, group, ())

        def drain_g(sem):
            pltpu.make_async_copy(
                ut_hbm.at[pl.ds(0, WIN)],
                rows_v.at[pl.ds(0, WIN)], sem).wait()

        def fire_write(out, w):
            pltpu.async_copy(rows_v.at[pl.ds(w * WIN, WIN)],
                             out.at[pl.ds(base + w * WIN, WIN)], wsem)

        def drain_w():
            pltpu.make_async_copy(
                out_u.at[pl.ds(0, WIN)], rows_v.at[pl.ds(0, WIN)], wsem).wait()

        gsems = (gsem_a, gsem_b)
        for t, (tbl, out) in enumerate(((ut_hbm, out_u), (it_hbm, out_i))):
            ioff = t * B_PER_W
            for w in range(N_WIN):
                fire_gathers(tbl, ioff, w, gsems[w % 2])
                if w >= 1:
                    drain_g(gsems[(w - 1) % 2])
                    fire_write(out, w - 1)
            drain_g(gsems[(N_WIN - 1) % 2])
            fire_write(out, N_WIN - 1)
            for _ in range(N_WIN):
                drain_w()

    return gather2


def kernel(query_users, query_items, user_table, item_table):
    if query_users.ndim > 1:
        query_users = jnp.squeeze(query_users, axis=0)
    if query_items.ndim > 1:
        query_items = jnp.squeeze(query_items, axis=0)
    return _build()(query_users.astype(jnp.int32),
                    query_items.astype(jnp.int32),
                    user_table, item_table)
